# Initial kernel scaffold; baseline (speedup 1.0000x reference)
import jax
import jax.numpy as jnp
from jax.experimental import pallas as pl


def kernel(sample, entity_embedding, relation_embedding):
    raise NotImplementedError("write your pallas kernel here")


# speedup vs baseline: 1.3135x; 1.3135x over previous
import functools

import jax
import jax.numpy as jnp
from jax import lax
from jax.experimental import pallas as pl
from jax.experimental.pallas import tpu as pltpu
from jax.experimental.pallas import tpu_sc as plsc

_NENTITY = 1000000
_NRELATION = 100000
_HIDDEN = 128
_ENT_DIM = 2 * _HIDDEN
_GAMMA = 12.0
_EPSILON = 2.0
_EMB_RANGE = (_GAMMA + _EPSILON) / _HIDDEN
_PI = 3.14159265358979323846
_PHASE_SCALE = _PI / _EMB_RANGE
_BATCH = 4096

_NC, _NS, _L = 2, 16, 16
_NW = _NC * _NS
_BPW = _BATCH // _NW
_CHUNKS = _HIDDEN // _L

_COS_C = (0.9999994437071157, -0.499995582285537, 0.0416610335190532,
          -0.001386274996095396, 2.425322988965727e-05, -2.2194129820797736e-07)
_SIN_C = (0.9999999562150126, -0.16666631918857602, 0.008332890671740594,
          -0.0001982075845429104, 2.712802783753961e-06, -2.087280676805603e-08)


def _poly_even(t, coeffs):
    acc = jnp.full((_L,), coeffs[-1], jnp.float32)
    for c in reversed(coeffs[:-1]):
        acc = acc * t + c
    return acc


_GATHER_DNUMS = lax.GatherDimensionNumbers(
    offset_dims=(), collapsed_slice_dims=(0,), start_index_map=(0,))


def _lane_shuffle(v, idx):
    return lax.gather(v, idx[:, None], _GATHER_DNUMS, slice_sizes=(1,),
                      mode=lax.GatherScatterMode.PROMISE_IN_BOUNDS)


def _rsqrt_nr(x):
    i = lax.bitcast_convert_type(x, jnp.int32)
    i = jnp.int32(0x5F3759DF) - lax.shift_right_logical(i, 1)
    y = lax.bitcast_convert_type(i, jnp.float32)
    xh = x * 0.5
    y = y * (1.5 - xh * y * y)
    y = y * (1.5 - xh * y * y)
    return y


def _sc_body(hidx_hbm, ridx_hbm, tidx_hbm, truef_hbm, ent_hbm, rel_hbm,
             out_hbm, hidx_v, ridx_v, tidx_v, truef_v, h_rows, t_rows,
             r_rows, score_v, sem_h, sem_t, sem_r):
    wid = lax.axis_index("s") * _NC + lax.axis_index("c")
    base = wid * _BPW

    pltpu.sync_copy(hidx_hbm.at[pl.ds(base, _BPW)], hidx_v)
    pltpu.sync_copy(tidx_hbm.at[pl.ds(base, _BPW)], tidx_v)
    pltpu.sync_copy(ridx_hbm.at[pl.ds(base, _BPW)], ridx_v)
    pltpu.sync_copy(truef_hbm.at[pl.ds(base, _BPW)], truef_v)

    cp_h = pltpu.async_copy(ent_hbm.at[hidx_v], h_rows, sem_h)
    cp_t = pltpu.async_copy(ent_hbm.at[tidx_v], t_rows, sem_t)
    cp_r = pltpu.async_copy(rel_hbm.at[ridx_v], r_rows, sem_r)
    cp_h.wait()
    cp_t.wait()
    cp_r.wait()

    lane = lax.iota(jnp.int32, _L)

    def sample_body(b, score_vec):
        acc = jnp.zeros((_L,), jnp.float32)
        for c in range(_CHUNKS):
            lo = pl.ds(c * _L, _L)
            hi = pl.ds(_HIDDEN + c * _L, _L)
            re_h = h_rows[b, lo]
            im_h = h_rows[b, hi]
            re_t = t_rows[b, lo]
            im_t = t_rows[b, hi]
            ph = r_rows[b, lo] * _PHASE_SCALE
            t2 = ph * ph
            cosv = _poly_even(t2, _COS_C)
            sinv = ph * _poly_even(t2, _SIN_C)
            re_s = re_h * cosv - im_h * sinv - re_t
            im_s = re_h * sinv + im_h * cosv - im_t
            m2 = re_s * re_s + im_s * im_s + 1e-24
            acc = acc + m2 * _rsqrt_nr(m2)
        for sh in (8, 4, 2, 1):
            acc = acc + _lane_shuffle(acc, lane ^ sh)
        j = lax.rem(b, _L)
        score_vec = jnp.where(lane == j, _GAMMA - acc, score_vec)

        @pl.when(j == _L - 1)
        def _():
            gbase = b - (_L - 1)
            sl = pl.ds(gbase, _L)
            score_v[sl] = score_vec * truef_v[sl]

        return score_vec

    lax.fori_loop(0, _BPW, sample_body, jnp.zeros((_L,), jnp.float32))
    pltpu.sync_copy(score_v, out_hbm.at[pl.ds(base, _BPW)])


@jax.jit
def _sc_score(hidx, ridx, tidx, truef, ent, rel):
    mesh = plsc.VectorSubcoreMesh(core_axis_name="c", subcore_axis_name="s")
    k = pl.kernel(
        _sc_body,
        out_type=jax.ShapeDtypeStruct((_BATCH,), jnp.float32),
        mesh=mesh,
        scratch_types=[
            pltpu.VMEM((_BPW,), jnp.int32),
            pltpu.VMEM((_BPW,), jnp.int32),
            pltpu.VMEM((_BPW,), jnp.int32),
            pltpu.VMEM((_BPW,), jnp.float32),
            pltpu.VMEM((_BPW, _ENT_DIM), jnp.float32),
            pltpu.VMEM((_BPW, _ENT_DIM), jnp.float32),
            pltpu.VMEM((_BPW, _HIDDEN), jnp.float32),
            pltpu.VMEM((_BPW,), jnp.float32),
            pltpu.SemaphoreType.DMA,
            pltpu.SemaphoreType.DMA,
            pltpu.SemaphoreType.DMA,
        ],
    )
    return k(hidx, ridx, tidx, truef, ent, rel)


def kernel(sample, entity_embedding, relation_embedding):
    hidx = sample[:, 0]
    ridx = sample[:, 1]
    tidx = sample[:, 2]
    truef = 1.0 - sample[:, 3].astype(jnp.float32)
    score = _sc_score(hidx, ridx, tidx, truef,
                      entity_embedding, relation_embedding)
    return (score[:, None], jnp.array(0.0, dtype=jnp.float32))


# speedup vs baseline: 1.4205x; 1.0815x over previous
import jax
import jax.numpy as jnp
from jax import lax
from jax.experimental import pallas as pl
from jax.experimental.pallas import tpu as pltpu
from jax.experimental.pallas import tpu_sc as plsc

_HIDDEN = 128
_ENT_DIM = 2 * _HIDDEN
_GAMMA = 12.0
_EPSILON = 2.0
_EMB_RANGE = (_GAMMA + _EPSILON) / _HIDDEN
_PI = 3.14159265358979323846
_PHASE_SCALE = _PI / _EMB_RANGE
_BATCH = 4096

_NC, _NS, _L = 2, 16, 16
_NW = _NC * _NS
_BPW = _BATCH // _NW
_CHUNKS = _HIDDEN // _L
_UNROLL = 4
_STRIDE = _BPW // _UNROLL

_COS_C = (0.9999710932182878, -0.4998375960856004, 0.04152230455016234,
          -0.0013441068677423887, 1.9065216086952955e-05)
_SIN_C = (0.9999972899501943, -0.16665146113624504, 0.008319843694976152,
          -0.000194241818811178, 2.22488813925666e-06)

_GATHER_DNUMS = lax.GatherDimensionNumbers(
    offset_dims=(), collapsed_slice_dims=(0,), start_index_map=(0,))


def _lane_shuffle(v, idx):
    return lax.gather(v, idx[:, None], _GATHER_DNUMS, slice_sizes=(1,),
                      mode=lax.GatherScatterMode.PROMISE_IN_BOUNDS)


def _poly5(t2, t4, c):
    return (c[0] + c[1] * t2) + t4 * ((c[2] + c[3] * t2) + c[4] * t4)


def _rsqrt_nr(x):
    i = lax.bitcast_convert_type(x, jnp.int32)
    i = jnp.int32(0x5F3759DF) - lax.shift_right_logical(i, 1)
    y = lax.bitcast_convert_type(i, jnp.float32)
    xh = 0.5 * x
    y = y * (1.5 - xh * y * y)
    return y * (1.5 - xh * y * y)


def _score_one(h_rows, t_rows, r_rows, b, lane):
    acc = jnp.zeros((_L,), jnp.float32)
    for c in range(_CHUNKS):
        lo = pl.ds(c * _L, _L)
        hi = pl.ds(_HIDDEN + c * _L, _L)
        re_h = h_rows[b, lo]
        im_h = h_rows[b, hi]
        re_t = t_rows[b, lo]
        im_t = t_rows[b, hi]
        ph = r_rows[b, lo] * _PHASE_SCALE
        t2 = ph * ph
        t4 = t2 * t2
        cosv = _poly5(t2, t4, _COS_C)
        sinv = ph * _poly5(t2, t4, _SIN_C)
        re_s = re_h * cosv - im_h * sinv - re_t
        im_s = re_h * sinv + im_h * cosv - im_t
        m2 = re_s * re_s + im_s * im_s
        acc = acc + m2 * _rsqrt_nr(m2)
    for sh in (8, 4, 2, 1):
        acc = acc + _lane_shuffle(acc, lane ^ sh)
    return _GAMMA - acc


def _sc_body(sample_t_hbm, ent_hbm, rel_hbm, out_hbm,
             hidx_v, ridx_v, tidx_v, tru_v, h_rows, t_rows, r_rows,
             score_v, sem_h, sem_t, sem_r):
    wid = lax.axis_index("s") * _NC + lax.axis_index("c")
    base = wid * _BPW
    sl_w = pl.ds(base, _BPW)

    pltpu.sync_copy(sample_t_hbm.at[0, sl_w], hidx_v)
    pltpu.sync_copy(sample_t_hbm.at[2, sl_w], tidx_v)
    pltpu.sync_copy(sample_t_hbm.at[1, sl_w], ridx_v)
    pltpu.sync_copy(sample_t_hbm.at[3, sl_w], tru_v)

    cp_h = pltpu.async_copy(ent_hbm.at[hidx_v], h_rows, sem_h)
    cp_t = pltpu.async_copy(ent_hbm.at[tidx_v], t_rows, sem_t)
    cp_r = pltpu.async_copy(rel_hbm.at[ridx_v], r_rows, sem_r)
    cp_h.wait()
    cp_t.wait()
    cp_r.wait()

    lane = lax.iota(jnp.int32, _L)

    def sample_body(b, svs):
        outs = [_score_one(h_rows, t_rows, r_rows, b + k * _STRIDE, lane)
                for k in range(_UNROLL)]
        j = jnp.bitwise_and(b, _L - 1)
        svs = tuple(jnp.where(lane == j, o, sv) for o, sv in zip(outs, svs))

        @pl.when(j == _L - 1)
        def _():
            gbase = b - (_L - 1)
            for k in range(_UNROLL):
                sl = pl.ds(gbase + k * _STRIDE, _L)
                truef = 1.0 - tru_v[sl].astype(jnp.float32)
                score_v[sl] = svs[k] * truef

        return svs

    zero = jnp.zeros((_L,), jnp.float32)
    lax.fori_loop(0, _STRIDE, sample_body, (zero,) * _UNROLL)
    pltpu.sync_copy(score_v, out_hbm.at[sl_w])


@jax.jit
def _sc_score(sample_t, ent, rel):
    mesh = plsc.VectorSubcoreMesh(core_axis_name="c", subcore_axis_name="s")
    k = pl.kernel(
        _sc_body,
        out_type=jax.ShapeDtypeStruct((_BATCH,), jnp.float32),
        mesh=mesh,
        scratch_types=[
            pltpu.VMEM((_BPW,), jnp.int32),
            pltpu.VMEM((_BPW,), jnp.int32),
            pltpu.VMEM((_BPW,), jnp.int32),
            pltpu.VMEM((_BPW,), jnp.int32),
            pltpu.VMEM((_BPW, _ENT_DIM), jnp.float32),
            pltpu.VMEM((_BPW, _ENT_DIM), jnp.float32),
            pltpu.VMEM((_BPW, _HIDDEN), jnp.float32),
            pltpu.VMEM((_BPW,), jnp.float32),
            pltpu.SemaphoreType.DMA,
            pltpu.SemaphoreType.DMA,
            pltpu.SemaphoreType.DMA,
        ],
    )
    return k(sample_t, ent, rel)


def kernel(sample, entity_embedding, relation_embedding):
    score = _sc_score(sample.T, entity_embedding, relation_embedding)
    return (score[:, None], jnp.array(0.0, dtype=jnp.float32))


# speedup vs baseline: 1.4600x; 1.0278x over previous
import jax
import jax.numpy as jnp
from jax import lax
from jax.experimental import pallas as pl
from jax.experimental.pallas import tpu as pltpu
from jax.experimental.pallas import tpu_sc as plsc

_HIDDEN = 128
_ENT_DIM = 2 * _HIDDEN
_GAMMA = 12.0
_EPSILON = 2.0
_EMB_RANGE = (_GAMMA + _EPSILON) / _HIDDEN
_PI = 3.14159265358979323846
_PHASE_SCALE = _PI / _EMB_RANGE
_BATCH = 4096

_NC, _NS, _L = 2, 16, 16
_NW = _NC * _NS
_BPW = _BATCH // _NW
_SEG = _BPW // 2
_CHUNKS = _HIDDEN // _L
_UNROLL = 4
_STRIDE = _SEG // _UNROLL

_COS_RAW = (0.9999710932182878, -0.4998375960856004, 0.04152230455016234,
            -0.0013441068677423887, 1.9065216086952955e-05)
_SIN_RAW = (0.9999972899501943, -0.16665146113624504, 0.008319843694976152,
            -0.000194241818811178, 2.22488813925666e-06)
_PS2 = _PHASE_SCALE * _PHASE_SCALE
_COS_C = tuple(c * _PS2 ** k for k, c in enumerate(_COS_RAW))
_SIN_C = tuple(_PHASE_SCALE * c * _PS2 ** k for k, c in enumerate(_SIN_RAW))

_GATHER_DNUMS = lax.GatherDimensionNumbers(
    offset_dims=(), collapsed_slice_dims=(0,), start_index_map=(0,))


def _lane_shuffle(v, idx):
    return lax.gather(v, idx[:, None], _GATHER_DNUMS, slice_sizes=(1,),
                      mode=lax.GatherScatterMode.PROMISE_IN_BOUNDS)


def _poly5(t2, t4, c):
    return (c[0] + c[1] * t2) + t4 * ((c[2] + c[3] * t2) + c[4] * t4)


def _rsqrt_nr(x):
    i = lax.bitcast_convert_type(x, jnp.int32)
    i = jnp.int32(0x5F3759DF) - lax.shift_right_logical(i, 1)
    y = lax.bitcast_convert_type(i, jnp.float32)
    xh = 0.5 * x
    y = y * (1.5 - xh * y * y)
    return y * (1.5 - xh * y * y)


def _score_one(h_rows, t_rows, r_rows, b, lane):
    acc = jnp.zeros((_L,), jnp.float32)
    for c in range(_CHUNKS):
        lo = pl.ds(c * _L, _L)
        hi = pl.ds(_HIDDEN + c * _L, _L)
        re_h = h_rows[b, lo]
        im_h = h_rows[b, hi]
        re_t = t_rows[b, lo]
        im_t = t_rows[b, hi]
        r = r_rows[b, lo]
        t2 = r * r
        t4 = t2 * t2
        cosv = _poly5(t2, t4, _COS_C)
        sinv = r * _poly5(t2, t4, _SIN_C)
        re_s = re_h * cosv - im_h * sinv - re_t
        im_s = re_h * sinv + im_h * cosv - im_t
        m2 = re_s * re_s + im_s * im_s
        acc = acc + m2 * _rsqrt_nr(m2)
    for sh in (8, 4, 2, 1):
        acc = acc + _lane_shuffle(acc, lane ^ sh)
    return _GAMMA - acc


def _sc_body(sample_t_hbm, ent_hbm, rel_hbm, out_hbm,
             hidx_v, ridx_v, tidx_v, tru_v,
             h0, h1, t0, t1, r0, r1,
             score_v, sem_i, sem0, sem1):
    wid = lax.axis_index("s") * _NC + lax.axis_index("c")
    base = wid * _BPW
    sl_w = pl.ds(base, _BPW)

    ci_h = pltpu.async_copy(sample_t_hbm.at[0, sl_w], hidx_v, sem_i)
    ci_t = pltpu.async_copy(sample_t_hbm.at[2, sl_w], tidx_v, sem_i)
    ci_r = pltpu.async_copy(sample_t_hbm.at[1, sl_w], ridx_v, sem_i)
    ci_u = pltpu.async_copy(sample_t_hbm.at[3, sl_w], tru_v, sem_i)
    ci_h.wait()
    ci_t.wait()
    ci_r.wait()

    lo_s = pl.ds(0, _SEG)
    hi_s = pl.ds(_SEG, _SEG)
    cp = [
        pltpu.async_copy(ent_hbm.at[hidx_v.at[lo_s]], h0, sem0),
        pltpu.async_copy(ent_hbm.at[tidx_v.at[lo_s]], t0, sem0),
        pltpu.async_copy(rel_hbm.at[ridx_v.at[lo_s]], r0, sem0),
        pltpu.async_copy(ent_hbm.at[hidx_v.at[hi_s]], h1, sem1),
        pltpu.async_copy(ent_hbm.at[tidx_v.at[hi_s]], t1, sem1),
        pltpu.async_copy(rel_hbm.at[ridx_v.at[hi_s]], r1, sem1),
    ]
    ci_u.wait()

    lane = lax.iota(jnp.int32, _L)

    for seg, (hh, tt, rr) in enumerate(((h0, t0, r0), (h1, t1, r1))):
        for c in cp[3 * seg:3 * seg + 3]:
            c.wait()

        def sample_body(b, svs, hh=hh, tt=tt, rr=rr, seg=seg):
            outs = [_score_one(hh, tt, rr, b + k * _STRIDE, lane)
                    for k in range(_UNROLL)]
            svs = tuple(jnp.where(lane == b, o, sv)
                        for o, sv in zip(outs, svs))

            @pl.when(b == _L - 1)
            def _():
                for k in range(_UNROLL):
                    sl = pl.ds(seg * _SEG + k * _STRIDE, _L)
                    truef = 1.0 - tru_v[sl].astype(jnp.float32)
                    score_v[sl] = svs[k] * truef

            return svs

        zero = jnp.zeros((_L,), jnp.float32)
        lax.fori_loop(0, _STRIDE, sample_body, (zero,) * _UNROLL)

    pltpu.sync_copy(score_v, out_hbm.at[sl_w])


@jax.jit
def _sc_score(sample_t, ent, rel):
    mesh = plsc.VectorSubcoreMesh(core_axis_name="c", subcore_axis_name="s")
    k = pl.kernel(
        _sc_body,
        out_type=jax.ShapeDtypeStruct((_BATCH,), jnp.float32),
        mesh=mesh,
        scratch_types=[
            pltpu.VMEM((_BPW,), jnp.int32),
            pltpu.VMEM((_BPW,), jnp.int32),
            pltpu.VMEM((_BPW,), jnp.int32),
            pltpu.VMEM((_BPW,), jnp.int32),
            pltpu.VMEM((_SEG, _ENT_DIM), jnp.float32),
            pltpu.VMEM((_SEG, _ENT_DIM), jnp.float32),
            pltpu.VMEM((_SEG, _ENT_DIM), jnp.float32),
            pltpu.VMEM((_SEG, _ENT_DIM), jnp.float32),
            pltpu.VMEM((_SEG, _HIDDEN), jnp.float32),
            pltpu.VMEM((_SEG, _HIDDEN), jnp.float32),
            pltpu.VMEM((_BPW,), jnp.float32),
            pltpu.SemaphoreType.DMA,
            pltpu.SemaphoreType.DMA,
            pltpu.SemaphoreType.DMA,
        ],
    )
    return k(sample_t, ent, rel)


def kernel(sample, entity_embedding, relation_embedding):
    score = _sc_score(sample.T, entity_embedding, relation_embedding)
    return (score[:, None], jnp.array(0.0, dtype=jnp.float32))
